# baseline (speedup 1.0000x reference)
import functools

import jax
import jax.numpy as jnp
from jax import lax
from jax.experimental import pallas as pl
from jax.experimental.pallas import tpu as pltpu
from jax.experimental.pallas import tpu_sc as plsc

B = 64
N = 1000
DIN = 128
DOUT = 128
EMB = 16
M = 64
K = 32

TN = 8


NW = 32
ROWS = B * M * K
RPW = ROWS // NW
CHUNK = 256


@functools.partial(
    pl.kernel,
    mesh=plsc.VectorSubcoreMesh(core_axis_name="c", subcore_axis_name="s"),
    out_type=[
        jax.ShapeDtypeStruct((ROWS, DIN), jnp.float32),
        jax.ShapeDtypeStruct((ROWS, DIN), jnp.float32),
    ],
    scratch_types=[
        pltpu.VMEM((CHUNK,), jnp.int32),
        pltpu.VMEM((CHUNK, DIN), jnp.float32),
        pltpu.VMEM((CHUNK, DIN), jnp.float32),
        pltpu.SemaphoreType.DMA,
        pltpu.SemaphoreType.DMA,
    ],
)
def _gather_sc(xflat, nvflat, gidx, selx, selnv, idx_v, rows_v, nv_v,
               sem1, sem2):
    sid = lax.axis_index("s")
    cid = lax.axis_index("c")
    base = (sid * 2 + cid) * RPW

    def body(ci, carry):
        off = base + ci * CHUNK
        pltpu.sync_copy(gidx.at[pl.ds(off, CHUNK)], idx_v)
        c1 = pltpu.async_copy(xflat.at[idx_v], rows_v, sem1)
        c2 = pltpu.async_copy(nvflat.at[idx_v], nv_v, sem2)
        c1.wait()
        c2.wait()
        pltpu.sync_copy(rows_v, selx.at[pl.ds(off, CHUNK)])
        pltpu.sync_copy(nv_v, selnv.at[pl.ds(off, CHUNK)])
        return carry

    lax.fori_loop(0, RPW // CHUNK, body, 0)


R_TOPK = B * M
RB = 256


def _topk_body(log_ref, idx_ref):
    lg = log_ref[...]
    s = lax.bitcast_convert_type(lg, jnp.int32)
    m = lax.shift_right_arithmetic(s, 31) & jnp.int32(0x7FFFFFFF)
    key = lax.bitcast_convert_type(s ^ m, jnp.uint32) ^ jnp.uint32(0x80000000)
    t = jnp.zeros((RB, 1), jnp.uint32)
    for bit in range(31, -1, -1):
        t2 = t | jnp.uint32(1 << bit)
        cnt = jnp.sum((key >= t2).astype(jnp.int32), axis=1, keepdims=True)
        t = jnp.where(cnt >= K, t2, t)
    strict = (key > t).astype(jnp.int32)
    ties = (key == t).astype(jnp.int32)
    iot = lax.broadcasted_iota(jnp.int32, (N, N), 0)
    jot = lax.broadcasted_iota(jnp.int32, (N, N), 1)
    tril = (jot <= iot).astype(jnp.float32)
    cum_t = lax.dot_general(ties.astype(jnp.float32), tril,
                            (((1,), (1,)), ((), ())),
                            preferred_element_type=jnp.float32)
    c1 = jnp.sum(strict, axis=1, keepdims=True)
    sel = strict + ties * (cum_t <= (K - c1).astype(jnp.float32)
                           ).astype(jnp.int32)
    cum = lax.dot_general(sel.astype(jnp.float32), tril,
                          (((1,), (1,)), ((), ())),
                          preferred_element_type=jnp.float32)
    for k in range(K):
        idx_ref[:, k] = jnp.sum((cum <= k).astype(jnp.int32), axis=1)


def _topk_tc(logits2d):
    return pl.pallas_call(
        _topk_body,
        grid=(R_TOPK // RB,),
        in_specs=[pl.BlockSpec((RB, N), lambda i: (i, 0))],
        out_specs=pl.BlockSpec((RB, K), lambda i: (i, 0)),
        out_shape=jax.ShapeDtypeStruct((R_TOPK, K), jnp.int32),
    )(logits2d)


BM = 16


def _subgraph_body(sel_ref, selx_ref, out_ref):
    for i in range(BM):
        s = sel_ref[i]
        g = jax.nn.relu(lax.dot_general(s, s, (((1,), (1,)), ((), ())),
                                        preferred_element_type=jnp.float32))
        d = lax.rsqrt(jnp.sum(g, axis=1, keepdims=True))
        lpl = g * d * jnp.transpose(d)
        out_ref[i] = jnp.dot(lpl, selx_ref[i],
                             preferred_element_type=jnp.float32)


def _subgraph(selnv_flat, selx_flat):
    sel3 = selnv_flat.reshape(B * M, K, DIN)
    selx3 = selx_flat.reshape(B * M, K, DIN)
    return pl.pallas_call(
        _subgraph_body,
        grid=(B * M // BM,),
        in_specs=[
            pl.BlockSpec((BM, K, DIN), lambda i: (i, 0, 0)),
            pl.BlockSpec((BM, K, DIN), lambda i: (i, 0, 0)),
        ],
        out_specs=pl.BlockSpec((BM, K, DIN), lambda i: (i, 0, 0)),
        out_shape=jax.ShapeDtypeStruct((B * M, K, DIN), jnp.float32),
    )(sel3, selx3)


def _front_body(x_ref, ne0_ref, f1w_ref, f1b_ref, f2w_ref, f2b_ref,
                f3w_ref, f3b_ref, sel_ref, nv_ref, log_ref):
    cdims = (((1,), (1,)), ((), ()))
    xb = x_ref[0]
    h1 = jax.nn.sigmoid(
        lax.dot_general(xb, f1w_ref[...], cdims,
                        preferred_element_type=jnp.float32) + f1b_ref[...])
    h2 = jax.nn.sigmoid(
        lax.dot_general(h1, f2w_ref[...], cdims,
                        preferred_element_type=jnp.float32) + f2b_ref[...])
    filt = lax.dot_general(h2, f3w_ref[...], cdims,
                           preferred_element_type=jnp.float32) + f3b_ref[...]
    nv = ne0_ref[...] * filt
    nv_ref[0] = jnp.concatenate(
        [nv, jnp.zeros((N, DIN - EMB), jnp.float32)], axis=1)
    log_ref[0] = lax.dot_general(sel_ref[...], nv, cdims,
                                 preferred_element_type=jnp.float32)


def _front(x, ne0, fc1_w, fc1_b, fc2_w, fc2_b, fc3_w, fc3_b, sel_emb):
    return pl.pallas_call(
        _front_body,
        grid=(B,),
        in_specs=[
            pl.BlockSpec((1, N, DIN), lambda b: (b, 0, 0)),
            pl.BlockSpec((N, EMB), lambda b: (0, 0)),
            pl.BlockSpec(fc1_w.shape, lambda b: (0, 0)),
            pl.BlockSpec((1, fc1_b.shape[0]), lambda b: (0, 0)),
            pl.BlockSpec(fc2_w.shape, lambda b: (0, 0)),
            pl.BlockSpec((1, fc2_b.shape[0]), lambda b: (0, 0)),
            pl.BlockSpec(fc3_w.shape, lambda b: (0, 0)),
            pl.BlockSpec((1, fc3_b.shape[0]), lambda b: (0, 0)),
            pl.BlockSpec((M, EMB), lambda b: (0, 0)),
        ],
        out_specs=[
            pl.BlockSpec((1, N, DIN), lambda b: (b, 0, 0)),
            pl.BlockSpec((1, M, N), lambda b: (b, 0, 0)),
        ],
        out_shape=[
            jax.ShapeDtypeStruct((B, N, DIN), jnp.float32),
            jax.ShapeDtypeStruct((B, M, N), jnp.float32),
        ],
    )(x, ne0, fc1_w, fc1_b[None, :], fc2_w, fc2_b[None, :],
      fc3_w, fc3_b[None, :], sel_emb)


def _outstage_body(ne1_ref, wpa_ref, wpb_ref, bp_ref, x_ref, xg2_ref, out_ref):
    ne1 = ne1_ref[...]
    wa = jnp.dot(ne1, wpa_ref[...], preferred_element_type=jnp.float32)
    wb = jnp.dot(ne1, wpb_ref[...], preferred_element_type=jnp.float32)
    bias = jnp.dot(ne1, bp_ref[...], preferred_element_type=jnp.float32)
    for n in range(TN):
        wan = wa[n].reshape(DIN, DOUT)
        wbn = wb[n].reshape(DIN, DOUT)
        r = (jnp.dot(x_ref[:, n, :], wan, preferred_element_type=jnp.float32)
             + jnp.dot(xg2_ref[:, n, :], wbn, preferred_element_type=jnp.float32)
             + bias[n][None, :])
        out_ref[:, n, :] = r


def _outstage(x, x_g2, ne1, weights_pool, bias_pool):
    wpa = weights_pool[:, 0].reshape(EMB, DIN * DOUT)
    wpb = weights_pool[:, 1].reshape(EMB, DIN * DOUT)
    return pl.pallas_call(
        _outstage_body,
        grid=(N // TN,),
        in_specs=[
            pl.BlockSpec((TN, EMB), lambda i: (i, 0)),
            pl.BlockSpec((EMB, DIN * DOUT), lambda i: (0, 0)),
            pl.BlockSpec((EMB, DIN * DOUT), lambda i: (0, 0)),
            pl.BlockSpec((EMB, DOUT), lambda i: (0, 0)),
            pl.BlockSpec((B, TN, DIN), lambda i: (0, i, 0)),
            pl.BlockSpec((B, TN, DIN), lambda i: (0, i, 0)),
        ],
        out_specs=pl.BlockSpec((B, TN, DOUT), lambda i: (0, i, 0)),
        out_shape=jax.ShapeDtypeStruct((B, N, DOUT), jnp.float32),
    )(ne1, wpa, wpb, bias_pool, x, x_g2)


def kernel(x, node_embeddings, fc1_w, fc1_b, fc2_w, fc2_b, fc3_w, fc3_b,
           weights_pool, bias_pool, sel_emb):
    B_, N_, DIN_ = x.shape
    nv_pad, logits = _front(x, node_embeddings[0], fc1_w, fc1_b,
                            fc2_w, fc2_b, fc3_w, fc3_b, sel_emb)
    logits2d = logits.reshape(R_TOPK, N_)
    indices = _topk_tc(logits2d).reshape(B_, M, K)

    gidx = (indices + (jnp.arange(B_, dtype=jnp.int32) * N_)[:, None, None]
            ).reshape(ROWS)
    selx_flat, selnv_flat = _gather_sc(
        x.reshape(B_ * N_, DIN_), nv_pad.reshape(B_ * N_, DIN_), gidx)
    node_new = _subgraph(selnv_flat, selx_flat).reshape(B_, M * K, DIN_)
    idx_flat = indices.reshape(B_, M * K)
    b2 = jnp.broadcast_to(jnp.arange(B_)[:, None], idx_flat.shape)
    dict1 = jnp.zeros((B_, N_, DIN_), dtype=x.dtype).at[b2, idx_flat].add(node_new)
    counts = jnp.full((B_, N_), 1e-14, dtype=x.dtype).at[b2, idx_flat].add(1.0)
    x_g2 = dict1 / counts[..., None]

    return _outstage(x, x_g2, node_embeddings[1], weights_pool, bias_pool)


# speedup vs baseline: 1.8188x; 1.8188x over previous
import functools

import jax
import jax.numpy as jnp
from jax import lax
from jax.experimental import pallas as pl
from jax.experimental.pallas import tpu as pltpu
from jax.experimental.pallas import tpu_sc as plsc

B = 64
N = 1000
DIN = 128
DOUT = 128
EMB = 16
M = 64
K = 32

TN = 8


NW = 32
ROWS = B * M * K
RPW = ROWS // NW
CHUNK = 256


@functools.partial(
    pl.kernel,
    mesh=plsc.VectorSubcoreMesh(core_axis_name="c", subcore_axis_name="s"),
    out_type=[
        jax.ShapeDtypeStruct((ROWS, DIN), jnp.float32),
        jax.ShapeDtypeStruct((ROWS, DIN), jnp.float32),
    ],
    scratch_types=[
        pltpu.VMEM((CHUNK,), jnp.int32),
        pltpu.VMEM((CHUNK, DIN), jnp.float32),
        pltpu.VMEM((CHUNK, DIN), jnp.float32),
        pltpu.SemaphoreType.DMA,
        pltpu.SemaphoreType.DMA,
    ],
)
def _gather_sc(xflat, nvflat, gidx, selx, selnv, idx_v, rows_v, nv_v,
               sem1, sem2):
    sid = lax.axis_index("s")
    cid = lax.axis_index("c")
    base = (sid * 2 + cid) * RPW

    def body(ci, carry):
        off = base + ci * CHUNK
        pltpu.sync_copy(gidx.at[pl.ds(off, CHUNK)], idx_v)
        c1 = pltpu.async_copy(xflat.at[idx_v], rows_v, sem1)
        c2 = pltpu.async_copy(nvflat.at[idx_v], nv_v, sem2)
        c1.wait()
        c2.wait()
        pltpu.sync_copy(rows_v, selx.at[pl.ds(off, CHUNK)])
        pltpu.sync_copy(nv_v, selnv.at[pl.ds(off, CHUNK)])
        return carry

    lax.fori_loop(0, RPW // CHUNK, body, 0)


R_TOPK = B * M
RB = 256


def _topk_body(log_ref, idx_ref):
    lg = log_ref[...]
    s = lax.bitcast_convert_type(lg, jnp.int32)
    m = lax.shift_right_arithmetic(s, 31) & jnp.int32(0x7FFFFFFF)
    key = lax.bitcast_convert_type(s ^ m, jnp.uint32) ^ jnp.uint32(0x80000000)
    t = jnp.zeros((RB, 1), jnp.uint32)
    for bit in range(31, -1, -1):
        t2 = t | jnp.uint32(1 << bit)
        cnt = jnp.sum((key >= t2).astype(jnp.int32), axis=1, keepdims=True)
        t = jnp.where(cnt >= K, t2, t)
    strict = (key > t).astype(jnp.int32)
    ties = (key == t).astype(jnp.int32)
    iot = lax.broadcasted_iota(jnp.int32, (N, N), 0)
    jot = lax.broadcasted_iota(jnp.int32, (N, N), 1)
    tril = (jot <= iot).astype(jnp.float32)
    cum_t = lax.dot_general(ties.astype(jnp.float32), tril,
                            (((1,), (1,)), ((), ())),
                            preferred_element_type=jnp.float32)
    c1 = jnp.sum(strict, axis=1, keepdims=True)
    sel = strict + ties * (cum_t <= (K - c1).astype(jnp.float32)
                           ).astype(jnp.int32)
    cum = lax.dot_general(sel.astype(jnp.float32), tril,
                          (((1,), (1,)), ((), ())),
                          preferred_element_type=jnp.float32)
    for k in range(K):
        idx_ref[:, k] = jnp.sum((cum <= k).astype(jnp.int32), axis=1)


def _topk_tc(logits2d):
    return pl.pallas_call(
        _topk_body,
        grid=(R_TOPK // RB,),
        in_specs=[pl.BlockSpec((RB, N), lambda i: (i, 0))],
        out_specs=pl.BlockSpec((RB, K), lambda i: (i, 0)),
        out_shape=jax.ShapeDtypeStruct((R_TOPK, K), jnp.int32),
    )(logits2d)


def _front_body(x_ref, ne0_ref, f1w_ref, f1b_ref, f2w_ref, f2b_ref,
                f3w_ref, f3b_ref, sel_ref, nv_ref, log_ref):
    cdims = (((1,), (1,)), ((), ()))
    xb = x_ref[0]
    h1 = jax.nn.sigmoid(
        lax.dot_general(xb, f1w_ref[...], cdims,
                        preferred_element_type=jnp.float32) + f1b_ref[...])
    h2 = jax.nn.sigmoid(
        lax.dot_general(h1, f2w_ref[...], cdims,
                        preferred_element_type=jnp.float32) + f2b_ref[...])
    filt = lax.dot_general(h2, f3w_ref[...], cdims,
                           preferred_element_type=jnp.float32) + f3b_ref[...]
    nv = ne0_ref[...] * filt
    nv_ref[0] = jnp.concatenate(
        [nv, jnp.zeros((N, DIN - EMB), jnp.float32)], axis=1)
    log_ref[0] = lax.dot_general(sel_ref[...], nv, cdims,
                                 preferred_element_type=jnp.float32)


def _front(x, ne0, fc1_w, fc1_b, fc2_w, fc2_b, fc3_w, fc3_b, sel_emb):
    return pl.pallas_call(
        _front_body,
        grid=(B,),
        in_specs=[
            pl.BlockSpec((1, N, DIN), lambda b: (b, 0, 0)),
            pl.BlockSpec((N, EMB), lambda b: (0, 0)),
            pl.BlockSpec(fc1_w.shape, lambda b: (0, 0)),
            pl.BlockSpec((1, fc1_b.shape[0]), lambda b: (0, 0)),
            pl.BlockSpec(fc2_w.shape, lambda b: (0, 0)),
            pl.BlockSpec((1, fc2_b.shape[0]), lambda b: (0, 0)),
            pl.BlockSpec(fc3_w.shape, lambda b: (0, 0)),
            pl.BlockSpec((1, fc3_b.shape[0]), lambda b: (0, 0)),
            pl.BlockSpec((M, EMB), lambda b: (0, 0)),
        ],
        out_specs=[
            pl.BlockSpec((1, N, DIN), lambda b: (b, 0, 0)),
            pl.BlockSpec((1, M, N), lambda b: (b, 0, 0)),
        ],
        out_shape=[
            jax.ShapeDtypeStruct((B, N, DIN), jnp.float32),
            jax.ShapeDtypeStruct((B, M, N), jnp.float32),
        ],
    )(x, ne0, fc1_w, fc1_b[None, :], fc2_w, fc2_b[None, :],
      fc3_w, fc3_b[None, :], sel_emb)


def _outstage_body(ne1_ref, wpa_ref, wpb_ref, bp_ref, x_ref, xg2_ref, out_ref):
    ne1 = ne1_ref[...]
    wa = jnp.dot(ne1, wpa_ref[...], preferred_element_type=jnp.float32)
    wb = jnp.dot(ne1, wpb_ref[...], preferred_element_type=jnp.float32)
    bias = jnp.dot(ne1, bp_ref[...], preferred_element_type=jnp.float32)
    for n in range(TN):
        wan = wa[n].reshape(DIN, DOUT)
        wbn = wb[n].reshape(DIN, DOUT)
        r = (jnp.dot(x_ref[:, n, :], wan, preferred_element_type=jnp.float32)
             + jnp.dot(xg2_ref[:, n, :], wbn, preferred_element_type=jnp.float32)
             + bias[n][None, :])
        out_ref[:, n, :] = r


def _outstage(x, x_g2, ne1, weights_pool, bias_pool):
    wpa = weights_pool[:, 0].reshape(EMB, DIN * DOUT)
    wpb = weights_pool[:, 1].reshape(EMB, DIN * DOUT)
    return pl.pallas_call(
        _outstage_body,
        grid=(N // TN,),
        in_specs=[
            pl.BlockSpec((TN, EMB), lambda i: (i, 0)),
            pl.BlockSpec((EMB, DIN * DOUT), lambda i: (0, 0)),
            pl.BlockSpec((EMB, DIN * DOUT), lambda i: (0, 0)),
            pl.BlockSpec((EMB, DOUT), lambda i: (0, 0)),
            pl.BlockSpec((B, TN, DIN), lambda i: (0, i, 0)),
            pl.BlockSpec((B, TN, DIN), lambda i: (0, i, 0)),
        ],
        out_specs=pl.BlockSpec((B, TN, DOUT), lambda i: (0, i, 0)),
        out_shape=jax.ShapeDtypeStruct((B, N, DOUT), jnp.float32),
    )(ne1, wpa, wpb, bias_pool, x, x_g2)


def kernel(x, node_embeddings, fc1_w, fc1_b, fc2_w, fc2_b, fc3_w, fc3_b,
           weights_pool, bias_pool, sel_emb):
    B_, N_, DIN_ = x.shape
    nv_pad, logits = _front(x, node_embeddings[0], fc1_w, fc1_b,
                            fc2_w, fc2_b, fc3_w, fc3_b, sel_emb)
    logits2d = logits.reshape(R_TOPK, N_)
    indices = _topk_tc(logits2d).reshape(B_, M, K)

    gidx = (indices + (jnp.arange(B_, dtype=jnp.int32) * N_)[:, None, None]
            ).reshape(ROWS)
    selx_flat, selnv_flat = _gather_sc(
        x.reshape(B_ * N_, DIN_), nv_pad.reshape(B_ * N_, DIN_), gidx)
    sel_x = selx_flat.reshape(B_, M, K, DIN_)
    selected = selnv_flat.reshape(B_, M, K, DIN_)

    graph = jax.nn.relu(jnp.einsum('bmkd,bmjd->bmkj', selected, selected))
    d = jnp.sum(graph, axis=-1) ** (-0.5)
    L = graph * d[..., :, None] * d[..., None, :]

    node_new = jnp.einsum('bmkj,bmjd->bmkd', L, sel_x).reshape(B_, M * K, DIN_)
    idx_flat = indices.reshape(B_, M * K)
    b2 = jnp.broadcast_to(jnp.arange(B_)[:, None], idx_flat.shape)
    dict1 = jnp.zeros((B_, N_, DIN_), dtype=x.dtype).at[b2, idx_flat].add(node_new)
    counts = jnp.full((B_, N_), 1e-14, dtype=x.dtype).at[b2, idx_flat].add(1.0)
    x_g2 = dict1 / counts[..., None]

    return _outstage(x, x_g2, node_embeddings[1], weights_pool, bias_pool)
